# Initial kernel scaffold; baseline (speedup 1.0000x reference)
import jax
import jax.numpy as jnp
from jax.experimental import pallas as pl


def kernel(x, edge_index, gin_W1, gin_b1, gin_W2, gin_b2, gin_W3, gin_b3, gin_eps, W1, b1, W2, b2, W3, b3):
    raise NotImplementedError("write your pallas kernel here")


# speedup vs baseline: 67.3763x; 67.3763x over previous
import jax
import jax.numpy as jnp
from jax import lax
from jax.experimental import pallas as pl
from jax.experimental.pallas import tpu as pltpu
from jax.experimental.pallas import tpu_sc as plsc

_LANES = 128
_KCH = 8
_NC = 2
_NS = 16
_NW = _NC * _NS
_D = 8


def _sc_agg(h, src2d, dst2d, zeros_init):
    N = h.shape[0]
    n_pad = zeros_init.shape[0]
    rows = src2d.shape[0]
    rpw = rows // _NW
    iters = rpw // (2 * _KCH)
    ch = -(-N // _NS)
    ch = -(-ch // 8) * 8
    tail = N - (_NS - 1) * ch
    rows_z = n_pad // _NS

    def body(h_hbm, src_hbm, dst_hbm, zero_hbm, out_hbm,
             srcv, dstv, msgv, h_sh, acc_sh, gsem, isem):
        c = lax.axis_index("c")
        s = lax.axis_index("s")
        wid = s * _NC + c

        @pl.when(s < _NS - 1)
        def _():
            pltpu.sync_copy(h_hbm.at[pl.ds(s * ch, ch)],
                            h_sh.at[pl.ds(s * ch, ch)])

        @pl.when(s == _NS - 1)
        def _():
            pltpu.sync_copy(h_hbm.at[pl.ds((_NS - 1) * ch, tail)],
                            h_sh.at[pl.ds((_NS - 1) * ch, tail)])

        pltpu.sync_copy(zero_hbm.at[pl.ds(s * rows_z, rows_z)],
                        acc_sh.at[pl.ds(s * rows_z, rows_z)])
        plsc.subcore_barrier()

        wbase = wid * rpw
        pltpu.sync_copy(src_hbm.at[pl.ds(wbase, _KCH)], srcv.at[pl.ds(0, _KCH)])
        pltpu.sync_copy(dst_hbm.at[pl.ds(wbase, _KCH)], dstv.at[pl.ds(0, _KCH)])

        def step(i, carry):
            base = wbase + i * (2 * _KCH)
            for half in range(2):
                off = half * _KCH
                nxt = (1 - half) * _KCH
                nbase = lax.min(base + off + _KCH, rows - _KCH)
                d1 = pltpu.async_copy(src_hbm.at[pl.ds(nbase, _KCH)],
                                      srcv.at[pl.ds(nxt, _KCH)], isem)
                d2 = pltpu.async_copy(dst_hbm.at[pl.ds(nbase, _KCH)],
                                      dstv.at[pl.ds(nxt, _KCH)], isem)
                gs = [pltpu.async_copy(h_sh.at[srcv.at[off + j]],
                                       msgv.at[off + j], gsem)
                      for j in range(_KCH)]
                for g in gs:
                    g.wait()
                for j in range(_KCH):
                    pltpu.sync_copy(msgv.at[off + j],
                                    acc_sh.at[dstv.at[off + j]], add=True)
                d1.wait()
                d2.wait()
            return carry

        lax.fori_loop(0, iters, step, 0)
        plsc.subcore_barrier()

        @pl.when(s < _NS - 1)
        def _():
            pltpu.sync_copy(acc_sh.at[pl.ds(s * ch, ch)],
                            out_hbm.at[pl.ds(c * N + s * ch, ch)])

        @pl.when(s == _NS - 1)
        def _():
            pltpu.sync_copy(acc_sh.at[pl.ds((_NS - 1) * ch, tail)],
                            out_hbm.at[pl.ds(c * N + (_NS - 1) * ch, tail)])

    mesh = plsc.VectorSubcoreMesh(core_axis_name="c", subcore_axis_name="s")
    f = pl.kernel(
        body,
        out_type=jax.ShapeDtypeStruct((2 * N, _D), jnp.float32),
        mesh=mesh,
        compiler_params=pltpu.CompilerParams(use_tc_tiling_on_sc=False),
        scratch_types=[
            pltpu.VMEM((2 * _KCH, _LANES), jnp.int32),
            pltpu.VMEM((2 * _KCH, _LANES), jnp.int32),
            pltpu.VMEM((2 * _KCH, _LANES, _D), jnp.float32),
            pltpu.VMEM_SHARED((N, _D), jnp.float32),
            pltpu.VMEM_SHARED((n_pad, _D), jnp.float32),
            pltpu.SemaphoreType.DMA,
            pltpu.SemaphoreType.DMA,
        ],
    )
    return f(h, src2d, dst2d, zeros_init)


_BN = 2000


def _sigmoid(x):
    return 1.0 / (1.0 + jnp.exp(-x))


def _tc_update(h, parts, W, b2d, eps2d):
    N = h.shape[0]
    nb = N // _BN

    def ubody(eps_ref, h_ref, p0_ref, p1_ref, W_ref, b_ref, o_ref):
        t = h_ref[...] * (1.0 + eps_ref[0, 0]) + p0_ref[...] + p1_ref[...]
        o_ref[...] = jnp.maximum(
            jnp.dot(t, W_ref[...], preferred_element_type=jnp.float32)
            + b_ref[...], 0.0)

    return pl.pallas_call(
        ubody,
        grid=(nb,),
        in_specs=[
            pl.BlockSpec(memory_space=pltpu.SMEM),
            pl.BlockSpec((_BN, _D), lambda i: (i, 0)),
            pl.BlockSpec((_BN, _D), lambda i: (i, 0)),
            pl.BlockSpec((_BN, _D), lambda i: (i + nb, 0)),
            pl.BlockSpec((_D, _D), lambda i: (0, 0)),
            pl.BlockSpec((1, _D), lambda i: (0, 0)),
        ],
        out_specs=pl.BlockSpec((_BN, _D), lambda i: (i, 0)),
        out_shape=jax.ShapeDtypeStruct((N, _D), jnp.float32),
    )(eps2d, h, parts, parts, W, b2d)


def _tc_final(h, parts, gW, gb2d, eps2d, W1, b1, W2, b2, W3, b3):
    N = h.shape[0]
    nb = N // _BN

    def fbody(eps_ref, h_ref, p0_ref, p1_ref, gW_ref, gb_ref,
              W1_ref, b1_ref, W2_ref, b2_ref, W3_ref, b3_ref, o_ref):
        t = h_ref[...] * (1.0 + eps_ref[0, 0]) + p0_ref[...] + p1_ref[...]
        t = jnp.maximum(
            jnp.dot(t, gW_ref[...], preferred_element_type=jnp.float32)
            + gb_ref[...], 0.0)
        o = _sigmoid(jnp.dot(t, W1_ref[...],
                             preferred_element_type=jnp.float32) + b1_ref[...])
        o = _sigmoid(jnp.dot(o, W2_ref[...],
                             preferred_element_type=jnp.float32) + b2_ref[...])
        o_ref[...] = _sigmoid(jnp.dot(o, W3_ref[...],
                                      preferred_element_type=jnp.float32)
                              + b3_ref[...])

    return pl.pallas_call(
        fbody,
        grid=(nb,),
        in_specs=[
            pl.BlockSpec(memory_space=pltpu.SMEM),
            pl.BlockSpec((_BN, _D), lambda i: (i, 0)),
            pl.BlockSpec((_BN, _D), lambda i: (i, 0)),
            pl.BlockSpec((_BN, _D), lambda i: (i + nb, 0)),
            pl.BlockSpec((_D, _D), lambda i: (0, 0)),
            pl.BlockSpec((1, _D), lambda i: (0, 0)),
            pl.BlockSpec((_D, 20), lambda i: (0, 0)),
            pl.BlockSpec((1, 20), lambda i: (0, 0)),
            pl.BlockSpec((20, 30), lambda i: (0, 0)),
            pl.BlockSpec((1, 30), lambda i: (0, 0)),
            pl.BlockSpec((30, 1), lambda i: (0, 0)),
            pl.BlockSpec((1, 1), lambda i: (0, 0)),
        ],
        out_specs=pl.BlockSpec((_BN, 1), lambda i: (i, 0)),
        out_shape=jax.ShapeDtypeStruct((N, 1), jnp.float32),
    )(eps2d, h, parts, parts, gW, gb2d,
      W1, b1.reshape(1, 20), W2, b2.reshape(1, 30), W3, b3.reshape(1, 1))


def kernel(x, edge_index, gin_W1, gin_b1, gin_W2, gin_b2, gin_W3, gin_b3,
           gin_eps, W1, b1, W2, b2, W3, b3):
    N = x.shape[0]
    E = edge_index.shape[1]
    src = edge_index[0].astype(jnp.int32)
    dst = edge_index[1].astype(jnp.int32)

    rows = -(-E // _LANES)
    rpw = -(-rows // _NW)
    rpw = -(-rpw // (2 * _KCH)) * (2 * _KCH)
    rows_pad = rpw * _NW
    epad = rows_pad * _LANES - E
    src2d = jnp.concatenate([src, jnp.zeros((epad,), jnp.int32)]).reshape(
        rows_pad, _LANES)
    dst2d = jnp.concatenate([dst, jnp.full((epad,), N, jnp.int32)]).reshape(
        rows_pad, _LANES)
    n_pad = -(-(N + 1) // 128) * 128
    zeros_init = jnp.zeros((n_pad, _D), jnp.float32)

    dpad = _D - x.shape[1]
    gWs = tuple(jnp.pad(W, ((0, dpad), (0, dpad)))
                for W in (gin_W1, gin_W2, gin_W3))
    gbs = tuple(jnp.pad(b, (0, dpad)).reshape(1, _D)
                for b in (gin_b1, gin_b2, gin_b3))

    h = jnp.pad(x, ((0, 0), (0, dpad)))
    for l in range(2):
        parts = _sc_agg(h, src2d, dst2d, zeros_init)
        h = _tc_update(h, parts, gWs[l], gbs[l], gin_eps[l].reshape(1, 1))
    parts = _sc_agg(h, src2d, dst2d, zeros_init)
    return _tc_final(h, parts, gWs[2], gbs[2], gin_eps[2].reshape(1, 1),
                     jnp.pad(W1, ((0, dpad), (0, 0))), b1, W2, b2, W3, b3)


# speedup vs baseline: 77.3243x; 1.1476x over previous
import jax
import jax.numpy as jnp
from jax import lax
from jax.experimental import pallas as pl
from jax.experimental.pallas import tpu as pltpu
from jax.experimental.pallas import tpu_sc as plsc

_CH = 1000
_NC = 2
_NS = 16
_NW = _NC * _NS
_D = 8


def _sc_agg(h, src, dst, zeros_init):
    N = h.shape[0]
    n_pad = zeros_init.shape[0]
    E = src.shape[0]
    epw = E // _NW
    iters = epw // (2 * _CH)
    ch = -(-N // _NS)
    ch = -(-ch // 8) * 8
    tail = N - (_NS - 1) * ch
    rows_z = n_pad // _NS

    def body(h_hbm, src_hbm, dst_hbm, zero_hbm, out_hbm,
             srcv, dstv, msgv, h_sh, acc_sh, gsem, isem, ssem):
        c = lax.axis_index("c")
        s = lax.axis_index("s")
        wid = s * _NC + c

        @pl.when(s < _NS - 1)
        def _():
            pltpu.sync_copy(h_hbm.at[pl.ds(s * ch, ch)],
                            h_sh.at[pl.ds(s * ch, ch)])

        @pl.when(s == _NS - 1)
        def _():
            pltpu.sync_copy(h_hbm.at[pl.ds((_NS - 1) * ch, tail)],
                            h_sh.at[pl.ds((_NS - 1) * ch, tail)])

        pltpu.sync_copy(zero_hbm.at[pl.ds(s * rows_z, rows_z)],
                        acc_sh.at[pl.ds(s * rows_z, rows_z)])
        plsc.subcore_barrier()

        wbase = wid * epw
        pltpu.sync_copy(src_hbm.at[pl.ds(wbase, _CH)], srcv.at[0])
        pltpu.sync_copy(dst_hbm.at[pl.ds(wbase, _CH)], dstv.at[0])

        def step(i, carry):
            base = wbase + i * (2 * _CH)
            for half in range(2):
                off = half
                nxt = 1 - half
                nbase = lax.min(base + (half + 1) * _CH, E - _CH)
                d1 = pltpu.async_copy(src_hbm.at[pl.ds(nbase, _CH)],
                                      srcv.at[nxt], isem)
                d2 = pltpu.async_copy(dst_hbm.at[pl.ds(nbase, _CH)],
                                      dstv.at[nxt], isem)

                pltpu.async_copy(h_sh.at[srcv.at[off]], msgv.at[off],
                                 gsem).wait()
                pltpu.sync_copy(msgv.at[off], acc_sh.at[dstv.at[off]],
                                add=True)
                d1.wait()
                d2.wait()
            return carry

        lax.fori_loop(0, iters, step, 0)
        plsc.subcore_barrier()

        @pl.when(s < _NS - 1)
        def _():
            pltpu.sync_copy(acc_sh.at[pl.ds(s * ch, ch)],
                            out_hbm.at[pl.ds(c * N + s * ch, ch)])

        @pl.when(s == _NS - 1)
        def _():
            pltpu.sync_copy(acc_sh.at[pl.ds((_NS - 1) * ch, tail)],
                            out_hbm.at[pl.ds(c * N + (_NS - 1) * ch, tail)])

    mesh = plsc.VectorSubcoreMesh(core_axis_name="c", subcore_axis_name="s")
    f = pl.kernel(
        body,
        out_type=jax.ShapeDtypeStruct((2 * N, _D), jnp.float32),
        mesh=mesh,
        compiler_params=pltpu.CompilerParams(use_tc_tiling_on_sc=False),
        scratch_types=[
            pltpu.VMEM((2, _CH), jnp.int32),
            pltpu.VMEM((2, _CH), jnp.int32),
            pltpu.VMEM((2, _CH, _D), jnp.float32),
            pltpu.VMEM_SHARED((N, _D), jnp.float32),
            pltpu.VMEM_SHARED((n_pad, _D), jnp.float32),
            pltpu.SemaphoreType.DMA,
            pltpu.SemaphoreType.DMA,
            pltpu.SemaphoreType.DMA,
        ],
    )
    return f(h, src, dst, zeros_init)


_BN = 2000


def _sigmoid(x):
    return 1.0 / (1.0 + jnp.exp(-x))


def _tc_update(h, parts, W, b2d, eps2d):
    N = h.shape[0]
    nb = N // _BN

    def ubody(eps_ref, h_ref, p0_ref, p1_ref, W_ref, b_ref, o_ref):
        t = h_ref[...] * (1.0 + eps_ref[0, 0]) + p0_ref[...] + p1_ref[...]
        o_ref[...] = jnp.maximum(
            jnp.dot(t, W_ref[...], preferred_element_type=jnp.float32)
            + b_ref[...], 0.0)

    return pl.pallas_call(
        ubody,
        grid=(nb,),
        in_specs=[
            pl.BlockSpec(memory_space=pltpu.SMEM),
            pl.BlockSpec((_BN, _D), lambda i: (i, 0)),
            pl.BlockSpec((_BN, _D), lambda i: (i, 0)),
            pl.BlockSpec((_BN, _D), lambda i: (i + nb, 0)),
            pl.BlockSpec((_D, _D), lambda i: (0, 0)),
            pl.BlockSpec((1, _D), lambda i: (0, 0)),
        ],
        out_specs=pl.BlockSpec((_BN, _D), lambda i: (i, 0)),
        out_shape=jax.ShapeDtypeStruct((N, _D), jnp.float32),
    )(eps2d, h, parts, parts, W, b2d)


def _tc_final(h, parts, gW, gb2d, eps2d, W1, b1, W2, b2, W3, b3):
    N = h.shape[0]
    nb = N // _BN

    def fbody(eps_ref, h_ref, p0_ref, p1_ref, gW_ref, gb_ref,
              W1_ref, b1_ref, W2_ref, b2_ref, W3_ref, b3_ref, o_ref):
        t = h_ref[...] * (1.0 + eps_ref[0, 0]) + p0_ref[...] + p1_ref[...]
        t = jnp.maximum(
            jnp.dot(t, gW_ref[...], preferred_element_type=jnp.float32)
            + gb_ref[...], 0.0)
        o = _sigmoid(jnp.dot(t, W1_ref[...],
                             preferred_element_type=jnp.float32) + b1_ref[...])
        o = _sigmoid(jnp.dot(o, W2_ref[...],
                             preferred_element_type=jnp.float32) + b2_ref[...])
        o_ref[...] = _sigmoid(jnp.dot(o, W3_ref[...],
                                      preferred_element_type=jnp.float32)
                              + b3_ref[...])

    return pl.pallas_call(
        fbody,
        grid=(nb,),
        in_specs=[
            pl.BlockSpec(memory_space=pltpu.SMEM),
            pl.BlockSpec((_BN, _D), lambda i: (i, 0)),
            pl.BlockSpec((_BN, _D), lambda i: (i, 0)),
            pl.BlockSpec((_BN, _D), lambda i: (i + nb, 0)),
            pl.BlockSpec((_D, _D), lambda i: (0, 0)),
            pl.BlockSpec((1, _D), lambda i: (0, 0)),
            pl.BlockSpec((_D, 20), lambda i: (0, 0)),
            pl.BlockSpec((1, 20), lambda i: (0, 0)),
            pl.BlockSpec((20, 30), lambda i: (0, 0)),
            pl.BlockSpec((1, 30), lambda i: (0, 0)),
            pl.BlockSpec((30, 1), lambda i: (0, 0)),
            pl.BlockSpec((1, 1), lambda i: (0, 0)),
        ],
        out_specs=pl.BlockSpec((_BN, 1), lambda i: (i, 0)),
        out_shape=jax.ShapeDtypeStruct((N, 1), jnp.float32),
    )(eps2d, h, parts, parts, gW, gb2d,
      W1, b1.reshape(1, 20), W2, b2.reshape(1, 30), W3, b3.reshape(1, 1))


def kernel(x, edge_index, gin_W1, gin_b1, gin_W2, gin_b2, gin_W3, gin_b3,
           gin_eps, W1, b1, W2, b2, W3, b3):
    N = x.shape[0]
    E = edge_index.shape[1]
    src = edge_index[0].astype(jnp.int32)
    dst = edge_index[1].astype(jnp.int32)
    assert E % (_NW * 2 * _CH) == 0, "edge count must split evenly"
    n_pad = -(-(N + 1) // 128) * 128
    zeros_init = jnp.zeros((n_pad, _D), jnp.float32)

    dpad = _D - x.shape[1]
    gWs = tuple(jnp.pad(W, ((0, dpad), (0, dpad)))
                for W in (gin_W1, gin_W2, gin_W3))
    gbs = tuple(jnp.pad(b, (0, dpad)).reshape(1, _D)
                for b in (gin_b1, gin_b2, gin_b3))

    h = jnp.pad(x, ((0, 0), (0, dpad)))
    for l in range(2):
        parts = _sc_agg(h, src, dst, zeros_init)
        h = _tc_update(h, parts, gWs[l], gbs[l], gin_eps[l].reshape(1, 1))
    parts = _sc_agg(h, src, dst, zeros_init)
    return _tc_final(h, parts, gWs[2], gbs[2], gin_eps[2].reshape(1, 1),
                     jnp.pad(W1, ((0, dpad), (0, 0))), b1, W2, b2, W3, b3)


# speedup vs baseline: 93.9130x; 1.2145x over previous
import jax
import jax.numpy as jnp
from jax import lax
from jax.experimental import pallas as pl
from jax.experimental.pallas import tpu as pltpu
from jax.experimental.pallas import tpu_sc as plsc

_CH = 1000
_NC = 2
_NS = 16
_NW = _NC * _NS
_D = 8


def _sc_agg(h, src, dst, zeros_init):
    N = h.shape[0]
    n_pad = zeros_init.shape[0]
    E = src.shape[0]
    epw = E // _NW
    iters = epw // (4 * _CH)
    ch = -(-N // _NS)
    ch = -(-ch // 8) * 8
    tail = N - (_NS - 1) * ch
    rows_z = n_pad // _NS

    def body(h_hbm, src_hbm, dst_hbm, zero_hbm, out_hbm,
             srcv, dstv, msgv, h_sh, acc_sh, gsem, isem, ssem):
        c = lax.axis_index("c")
        s = lax.axis_index("s")
        wid = s * _NC + c

        @pl.when(s < _NS - 1)
        def _():
            pltpu.sync_copy(h_hbm.at[pl.ds(s * ch, ch)],
                            h_sh.at[pl.ds(s * ch, ch)])

        @pl.when(s == _NS - 1)
        def _():
            pltpu.sync_copy(h_hbm.at[pl.ds((_NS - 1) * ch, tail)],
                            h_sh.at[pl.ds((_NS - 1) * ch, tail)])

        pltpu.sync_copy(zero_hbm.at[pl.ds(s * rows_z, rows_z)],
                        acc_sh.at[pl.ds(s * rows_z, rows_z)])
        plsc.subcore_barrier()

        wbase = wid * epw
        pltpu.sync_copy(src_hbm.at[pl.ds(wbase, _CH)], srcv.at[0])
        pltpu.sync_copy(dst_hbm.at[pl.ds(wbase, _CH)], dstv.at[0])

        def step(i, carry):
            base = wbase + i * (4 * _CH)
            for k in range(4):
                mi = k % 2
                nxt = (k + 1) % 4
                nbase = lax.min(base + (k + 1) * _CH, E - _CH)
                d1 = pltpu.async_copy(src_hbm.at[pl.ds(nbase, _CH)],
                                      srcv.at[nxt], isem)
                d2 = pltpu.async_copy(dst_hbm.at[pl.ds(nbase, _CH)],
                                      dstv.at[nxt], isem)

                if k >= 2:
                    pltpu.make_async_copy(h_hbm.at[pl.ds(0, _CH)],
                                          msgv.at[mi], ssem).wait()
                else:
                    @pl.when(i > 0)
                    def _():
                        pltpu.make_async_copy(h_hbm.at[pl.ds(0, _CH)],
                                              msgv.at[mi], ssem).wait()

                pltpu.async_copy(h_sh.at[srcv.at[k]], msgv.at[mi],
                                 gsem).wait()
                pltpu.async_copy(msgv.at[mi], acc_sh.at[dstv.at[k]],
                                 ssem, add=True)
                d1.wait()
                d2.wait()
            return carry

        lax.fori_loop(0, iters, step, 0)
        pltpu.make_async_copy(h_hbm.at[pl.ds(0, _CH)], msgv.at[0], ssem).wait()
        pltpu.make_async_copy(h_hbm.at[pl.ds(0, _CH)], msgv.at[1], ssem).wait()
        plsc.subcore_barrier()

        @pl.when(s < _NS - 1)
        def _():
            pltpu.sync_copy(acc_sh.at[pl.ds(s * ch, ch)],
                            out_hbm.at[pl.ds(c * N + s * ch, ch)])

        @pl.when(s == _NS - 1)
        def _():
            pltpu.sync_copy(acc_sh.at[pl.ds((_NS - 1) * ch, tail)],
                            out_hbm.at[pl.ds(c * N + (_NS - 1) * ch, tail)])

    mesh = plsc.VectorSubcoreMesh(core_axis_name="c", subcore_axis_name="s")
    f = pl.kernel(
        body,
        out_type=jax.ShapeDtypeStruct((2 * N, _D), jnp.float32),
        mesh=mesh,
        compiler_params=pltpu.CompilerParams(use_tc_tiling_on_sc=False),
        scratch_types=[
            pltpu.VMEM((4, _CH), jnp.int32),
            pltpu.VMEM((4, _CH), jnp.int32),
            pltpu.VMEM((2, _CH, _D), jnp.float32),
            pltpu.VMEM_SHARED((N, _D), jnp.float32),
            pltpu.VMEM_SHARED((n_pad, _D), jnp.float32),
            pltpu.SemaphoreType.DMA,
            pltpu.SemaphoreType.DMA,
            pltpu.SemaphoreType.DMA,
        ],
    )
    return f(h, src, dst, zeros_init)


_BN = 2000


def _sigmoid(x):
    return 1.0 / (1.0 + jnp.exp(-x))


def _tc_update(h, parts, W, b2d, eps2d):
    N = h.shape[0]
    nb = N // _BN

    def ubody(eps_ref, h_ref, p0_ref, p1_ref, W_ref, b_ref, o_ref):
        t = h_ref[...] * (1.0 + eps_ref[0, 0]) + p0_ref[...] + p1_ref[...]
        o_ref[...] = jnp.maximum(
            jnp.dot(t, W_ref[...], preferred_element_type=jnp.float32)
            + b_ref[...], 0.0)

    return pl.pallas_call(
        ubody,
        grid=(nb,),
        in_specs=[
            pl.BlockSpec(memory_space=pltpu.SMEM),
            pl.BlockSpec((_BN, _D), lambda i: (i, 0)),
            pl.BlockSpec((_BN, _D), lambda i: (i, 0)),
            pl.BlockSpec((_BN, _D), lambda i: (i + nb, 0)),
            pl.BlockSpec((_D, _D), lambda i: (0, 0)),
            pl.BlockSpec((1, _D), lambda i: (0, 0)),
        ],
        out_specs=pl.BlockSpec((_BN, _D), lambda i: (i, 0)),
        out_shape=jax.ShapeDtypeStruct((N, _D), jnp.float32),
    )(eps2d, h, parts, parts, W, b2d)


def _tc_final(h, parts, gW, gb2d, eps2d, W1, b1, W2, b2, W3, b3):
    N = h.shape[0]
    nb = N // _BN

    def fbody(eps_ref, h_ref, p0_ref, p1_ref, gW_ref, gb_ref,
              W1_ref, b1_ref, W2_ref, b2_ref, W3_ref, b3_ref, o_ref):
        t = h_ref[...] * (1.0 + eps_ref[0, 0]) + p0_ref[...] + p1_ref[...]
        t = jnp.maximum(
            jnp.dot(t, gW_ref[...], preferred_element_type=jnp.float32)
            + gb_ref[...], 0.0)
        o = _sigmoid(jnp.dot(t, W1_ref[...],
                             preferred_element_type=jnp.float32) + b1_ref[...])
        o = _sigmoid(jnp.dot(o, W2_ref[...],
                             preferred_element_type=jnp.float32) + b2_ref[...])
        o_ref[...] = _sigmoid(jnp.dot(o, W3_ref[...],
                                      preferred_element_type=jnp.float32)
                              + b3_ref[...])

    return pl.pallas_call(
        fbody,
        grid=(nb,),
        in_specs=[
            pl.BlockSpec(memory_space=pltpu.SMEM),
            pl.BlockSpec((_BN, _D), lambda i: (i, 0)),
            pl.BlockSpec((_BN, _D), lambda i: (i, 0)),
            pl.BlockSpec((_BN, _D), lambda i: (i + nb, 0)),
            pl.BlockSpec((_D, _D), lambda i: (0, 0)),
            pl.BlockSpec((1, _D), lambda i: (0, 0)),
            pl.BlockSpec((_D, 20), lambda i: (0, 0)),
            pl.BlockSpec((1, 20), lambda i: (0, 0)),
            pl.BlockSpec((20, 30), lambda i: (0, 0)),
            pl.BlockSpec((1, 30), lambda i: (0, 0)),
            pl.BlockSpec((30, 1), lambda i: (0, 0)),
            pl.BlockSpec((1, 1), lambda i: (0, 0)),
        ],
        out_specs=pl.BlockSpec((_BN, 1), lambda i: (i, 0)),
        out_shape=jax.ShapeDtypeStruct((N, 1), jnp.float32),
    )(eps2d, h, parts, parts, gW, gb2d,
      W1, b1.reshape(1, 20), W2, b2.reshape(1, 30), W3, b3.reshape(1, 1))


def kernel(x, edge_index, gin_W1, gin_b1, gin_W2, gin_b2, gin_W3, gin_b3,
           gin_eps, W1, b1, W2, b2, W3, b3):
    N = x.shape[0]
    E = edge_index.shape[1]
    src = edge_index[0].astype(jnp.int32)
    dst = edge_index[1].astype(jnp.int32)
    assert E % (_NW * 2 * _CH) == 0, "edge count must split evenly"
    n_pad = -(-(N + 1) // 128) * 128
    zeros_init = jnp.zeros((n_pad, _D), jnp.float32)

    dpad = _D - x.shape[1]
    gWs = tuple(jnp.pad(W, ((0, dpad), (0, dpad)))
                for W in (gin_W1, gin_W2, gin_W3))
    gbs = tuple(jnp.pad(b, (0, dpad)).reshape(1, _D)
                for b in (gin_b1, gin_b2, gin_b3))

    h = jnp.pad(x, ((0, 0), (0, dpad)))
    for l in range(2):
        parts = _sc_agg(h, src, dst, zeros_init)
        h = _tc_update(h, parts, gWs[l], gbs[l], gin_eps[l].reshape(1, 1))
    parts = _sc_agg(h, src, dst, zeros_init)
    return _tc_final(h, parts, gWs[2], gbs[2], gin_eps[2].reshape(1, 1),
                     jnp.pad(W1, ((0, dpad), (0, 0))), b1, W2, b2, W3, b3)


# speedup vs baseline: 152.9054x; 1.6282x over previous
import jax
import jax.numpy as jnp
from jax import lax
from jax.experimental import pallas as pl
from jax.experimental.pallas import tpu as pltpu
from jax.experimental.pallas import tpu_sc as plsc

_CH = 1000
_NC = 2
_NS = 16
_NW = _NC * _NS
_D = 8


def _sc_agg(h, src, dst, zeros_init):
    N = h.shape[0]
    n_pad = zeros_init.shape[0]
    E = src.shape[0]
    epw = E // _NW
    iters = epw // (4 * _CH)
    ch = -(-N // _NS)
    ch = -(-ch // 8) * 8
    tail = N - (_NS - 1) * ch
    rows_z = n_pad // _NS

    def body(h_hbm, src_hbm, dst_hbm, zero_hbm, out_hbm,
             srcv, dstv, msgv, h_sh, acc_sh, gsem, isem, ssem):
        c = lax.axis_index("c")
        s = lax.axis_index("s")
        wid = s * _NC + c

        @pl.when(s < _NS - 1)
        def _():
            pltpu.sync_copy(h_hbm.at[pl.ds(s * ch, ch)],
                            h_sh.at[pl.ds(s * ch, ch)])

        @pl.when(s == _NS - 1)
        def _():
            pltpu.sync_copy(h_hbm.at[pl.ds((_NS - 1) * ch, tail)],
                            h_sh.at[pl.ds((_NS - 1) * ch, tail)])

        pltpu.sync_copy(zero_hbm.at[pl.ds(s * rows_z, rows_z)],
                        acc_sh.at[pl.ds(s * rows_z, rows_z)])
        plsc.subcore_barrier()

        wbase = wid * epw
        pltpu.sync_copy(src_hbm.at[pl.ds(wbase, _CH)], srcv.at[0])
        pltpu.sync_copy(dst_hbm.at[pl.ds(wbase, _CH)], dstv.at[0])

        def step(i, carry):
            base = wbase + i * (4 * _CH)
            for k in range(4):
                mi = k % 2
                nxt = (k + 1) % 4
                nbase = lax.min(base + (k + 1) * _CH, E - _CH)
                d1 = pltpu.async_copy(src_hbm.at[pl.ds(nbase, _CH)],
                                      srcv.at[nxt], isem)
                d2 = pltpu.async_copy(dst_hbm.at[pl.ds(nbase, _CH)],
                                      dstv.at[nxt], isem)

                if k >= 2:
                    pltpu.make_async_copy(h_hbm.at[pl.ds(0, _CH)],
                                          msgv.at[mi], ssem).wait()
                else:
                    @pl.when(i > 0)
                    def _():
                        pltpu.make_async_copy(h_hbm.at[pl.ds(0, _CH)],
                                              msgv.at[mi], ssem).wait()

                pltpu.async_copy(h_sh.at[srcv.at[k]], msgv.at[mi],
                                 gsem).wait()
                pltpu.async_copy(msgv.at[mi], acc_sh.at[dstv.at[k]],
                                 ssem, add=True)
                d1.wait()
                d2.wait()
            return carry

        lax.fori_loop(0, iters, step, 0)
        pltpu.make_async_copy(h_hbm.at[pl.ds(0, _CH)], msgv.at[0], ssem).wait()
        pltpu.make_async_copy(h_hbm.at[pl.ds(0, _CH)], msgv.at[1], ssem).wait()
        plsc.subcore_barrier()

        @pl.when(s < _NS - 1)
        def _():
            pltpu.sync_copy(acc_sh.at[pl.ds(s * ch, ch)],
                            out_hbm.at[pl.ds(c * N + s * ch, ch)])

        @pl.when(s == _NS - 1)
        def _():
            pltpu.sync_copy(acc_sh.at[pl.ds((_NS - 1) * ch, tail)],
                            out_hbm.at[pl.ds(c * N + (_NS - 1) * ch, tail)])

    mesh = plsc.VectorSubcoreMesh(core_axis_name="c", subcore_axis_name="s")
    f = pl.kernel(
        body,
        out_type=jax.ShapeDtypeStruct((2 * N, _D), jnp.float32),
        mesh=mesh,
        compiler_params=pltpu.CompilerParams(use_tc_tiling_on_sc=False),
        scratch_types=[
            pltpu.VMEM((4, _CH), jnp.int32),
            pltpu.VMEM((4, _CH), jnp.int32),
            pltpu.VMEM((2, _CH, _D), jnp.float32),
            pltpu.VMEM_SHARED((N, _D), jnp.float32),
            pltpu.VMEM_SHARED((n_pad, _D), jnp.float32),
            pltpu.SemaphoreType.DMA,
            pltpu.SemaphoreType.DMA,
            pltpu.SemaphoreType.DMA,
        ],
    )
    return f(h, src, dst, zeros_init)


_BR = 6250


def _sigmoid(x):
    return 1.0 / (1.0 + jnp.exp(-x))


def _tc_update(h128, parts128, Wb, bb, eps2d):
    R = h128.shape[0]
    nb = R // _BR

    def ubody(eps_ref, h_ref, p0_ref, p1_ref, W_ref, b_ref, o_ref):
        t = h_ref[...] * (1.0 + eps_ref[0, 0]) + p0_ref[0] + p1_ref[0]
        o_ref[...] = jnp.maximum(
            jnp.dot(t, W_ref[...], preferred_element_type=jnp.float32)
            + b_ref[...], 0.0)

    return pl.pallas_call(
        ubody,
        grid=(nb,),
        in_specs=[
            pl.BlockSpec(memory_space=pltpu.SMEM),
            pl.BlockSpec((_BR, 128), lambda i: (i, 0)),
            pl.BlockSpec((1, _BR, 128), lambda i: (0, i, 0)),
            pl.BlockSpec((1, _BR, 128), lambda i: (1, i, 0)),
            pl.BlockSpec((128, 128), lambda i: (0, 0)),
            pl.BlockSpec((1, 128), lambda i: (0, 0)),
        ],
        out_specs=pl.BlockSpec((_BR, 128), lambda i: (i, 0)),
        out_shape=jax.ShapeDtypeStruct((R, 128), jnp.float32),
    )(eps2d, h128, parts128, parts128, Wb, bb)


def _tc_final(h128, parts128, gWb, gbb, eps2d, W1b, b1b, W2b, b2b, W3b, b3b):
    R = h128.shape[0]
    nb = R // _BR

    def fbody(eps_ref, h_ref, p0_ref, p1_ref, gW_ref, gb_ref,
              W1_ref, b1_ref, W2_ref, b2_ref, W3_ref, b3_ref, o_ref):
        t = h_ref[...] * (1.0 + eps_ref[0, 0]) + p0_ref[0] + p1_ref[0]
        t = jnp.maximum(
            jnp.dot(t, gW_ref[...], preferred_element_type=jnp.float32)
            + gb_ref[...], 0.0)
        o = _sigmoid(jnp.dot(t, W1_ref[...],
                             preferred_element_type=jnp.float32) + b1_ref[...])
        o = _sigmoid(jnp.dot(o, W2_ref[...],
                             preferred_element_type=jnp.float32) + b2_ref[...])
        o_ref[...] = _sigmoid(jnp.dot(o, W3_ref[...],
                                      preferred_element_type=jnp.float32)
                              + b3_ref[...])

    return pl.pallas_call(
        fbody,
        grid=(nb,),
        in_specs=[
            pl.BlockSpec(memory_space=pltpu.SMEM),
            pl.BlockSpec((_BR, 128), lambda i: (i, 0)),
            pl.BlockSpec((1, _BR, 128), lambda i: (0, i, 0)),
            pl.BlockSpec((1, _BR, 128), lambda i: (1, i, 0)),
            pl.BlockSpec((128, 128), lambda i: (0, 0)),
            pl.BlockSpec((1, 128), lambda i: (0, 0)),
            pl.BlockSpec((128, 320), lambda i: (0, 0)),
            pl.BlockSpec((1, 320), lambda i: (0, 0)),
            pl.BlockSpec((320, 480), lambda i: (0, 0)),
            pl.BlockSpec((1, 480), lambda i: (0, 0)),
            pl.BlockSpec((480, 16), lambda i: (0, 0)),
            pl.BlockSpec((1, 16), lambda i: (0, 0)),
        ],
        out_specs=pl.BlockSpec((_BR, 16), lambda i: (i, 0)),
        out_shape=jax.ShapeDtypeStruct((R, 16), jnp.float32),
    )(eps2d, h128, parts128, parts128, gWb, gbb,
      W1b, b1b, W2b, b2b, W3b, b3b)


def kernel(x, edge_index, gin_W1, gin_b1, gin_W2, gin_b2, gin_W3, gin_b3,
           gin_eps, W1, b1, W2, b2, W3, b3):
    N = x.shape[0]
    E = edge_index.shape[1]
    src = edge_index[0].astype(jnp.int32)
    dst = edge_index[1].astype(jnp.int32)
    assert E % (_NW * 4 * _CH) == 0, "edge count must split evenly"
    assert N % (16 * _BR) == 0
    n_pad = -(-(N + 1) // 128) * 128
    zeros_init = jnp.zeros((n_pad, _D), jnp.float32)
    R = N // 16
    eye16 = jnp.eye(16, dtype=jnp.float32)

    dpad = _D - x.shape[1]
    gWbs = tuple(jnp.kron(eye16, jnp.pad(W, ((0, dpad), (0, dpad))))
                 for W in (gin_W1, gin_W2, gin_W3))
    gbbs = tuple(jnp.tile(jnp.pad(b, (0, dpad)), 16).reshape(1, 128)
                 for b in (gin_b1, gin_b2, gin_b3))
    W1b = jnp.kron(eye16, jnp.pad(W1, ((0, dpad), (0, 0))))
    b1b = jnp.tile(b1, 16).reshape(1, 320)
    W2b = jnp.kron(eye16, W2)
    b2b = jnp.tile(b2, 16).reshape(1, 480)
    W3b = jnp.kron(eye16, W3)
    b3b = jnp.tile(b3, 16).reshape(1, 16)

    h = jnp.pad(x, ((0, 0), (0, dpad)))
    for l in range(2):
        parts = _sc_agg(h, src, dst, zeros_init)
        h128 = _tc_update(h.reshape(R, 128), parts.reshape(2, R, 128),
                          gWbs[l], gbbs[l], gin_eps[l].reshape(1, 1))
        h = h128.reshape(N, _D)
    parts = _sc_agg(h, src, dst, zeros_init)
    out16 = _tc_final(h.reshape(R, 128), parts.reshape(2, R, 128),
                      gWbs[2], gbbs[2], gin_eps[2].reshape(1, 1),
                      W1b, b1b, W2b, b2b, W3b, b3b)
    return out16.reshape(N, 1)


# speedup vs baseline: 158.7644x; 1.0383x over previous
import jax
import jax.numpy as jnp
from jax import lax
from jax.experimental import pallas as pl
from jax.experimental.pallas import tpu as pltpu
from jax.experimental.pallas import tpu_sc as plsc

_CH = 1000
_NC = 2
_NS = 16
_NW = _NC * _NS
_D = 8


def _sc_agg(h, ei, zeros_init):
    N = h.shape[0]
    n_pad = zeros_init.shape[0]
    E = ei.shape[1]
    epw = E // _NW
    iters = epw // (4 * _CH)
    ch = -(-N // _NS)
    ch = -(-ch // 8) * 8
    tail = N - (_NS - 1) * ch
    rows_z = n_pad // _NS

    def body(h_hbm, ei_hbm, zero_hbm, out_hbm,
             srcv, dstv, msgv, h_sh, acc_sh, gsem, isem, ssem):
        c = lax.axis_index("c")
        s = lax.axis_index("s")
        wid = s * _NC + c

        @pl.when(s < _NS - 1)
        def _():
            pltpu.sync_copy(h_hbm.at[pl.ds(s * ch, ch)],
                            h_sh.at[pl.ds(s * ch, ch)])

        @pl.when(s == _NS - 1)
        def _():
            pltpu.sync_copy(h_hbm.at[pl.ds((_NS - 1) * ch, tail)],
                            h_sh.at[pl.ds((_NS - 1) * ch, tail)])

        pltpu.sync_copy(zero_hbm.at[pl.ds(s * rows_z, rows_z)],
                        acc_sh.at[pl.ds(s * rows_z, rows_z)])
        plsc.subcore_barrier()

        wbase = wid * epw
        pltpu.sync_copy(ei_hbm.at[0, pl.ds(wbase, _CH)], srcv.at[0])
        pltpu.sync_copy(ei_hbm.at[1, pl.ds(wbase, _CH)], dstv.at[0])

        def step(i, carry):
            base = wbase + i * (4 * _CH)
            for k in range(4):
                mi = k % 2
                nxt = (k + 1) % 4
                nbase = lax.min(base + (k + 1) * _CH, E - _CH)
                d1 = pltpu.async_copy(ei_hbm.at[0, pl.ds(nbase, _CH)],
                                      srcv.at[nxt], isem)
                d2 = pltpu.async_copy(ei_hbm.at[1, pl.ds(nbase, _CH)],
                                      dstv.at[nxt], isem)

                if k >= 2:
                    pltpu.make_async_copy(h_hbm.at[pl.ds(0, _CH)],
                                          msgv.at[mi], ssem).wait()
                else:
                    @pl.when(i > 0)
                    def _():
                        pltpu.make_async_copy(h_hbm.at[pl.ds(0, _CH)],
                                              msgv.at[mi], ssem).wait()

                pltpu.async_copy(h_sh.at[srcv.at[k]], msgv.at[mi],
                                 gsem).wait()
                pltpu.async_copy(msgv.at[mi], acc_sh.at[dstv.at[k]],
                                 ssem, add=True)
                d1.wait()
                d2.wait()
            return carry

        lax.fori_loop(0, iters, step, 0)
        pltpu.make_async_copy(h_hbm.at[pl.ds(0, _CH)], msgv.at[0], ssem).wait()
        pltpu.make_async_copy(h_hbm.at[pl.ds(0, _CH)], msgv.at[1], ssem).wait()
        plsc.subcore_barrier()

        @pl.when(s < _NS - 1)
        def _():
            pltpu.sync_copy(acc_sh.at[pl.ds(s * ch, ch)],
                            out_hbm.at[pl.ds(c * N + s * ch, ch)])

        @pl.when(s == _NS - 1)
        def _():
            pltpu.sync_copy(acc_sh.at[pl.ds((_NS - 1) * ch, tail)],
                            out_hbm.at[pl.ds(c * N + (_NS - 1) * ch, tail)])

    mesh = plsc.VectorSubcoreMesh(core_axis_name="c", subcore_axis_name="s")
    f = pl.kernel(
        body,
        out_type=jax.ShapeDtypeStruct((2 * N, _D), jnp.float32),
        mesh=mesh,
        compiler_params=pltpu.CompilerParams(use_tc_tiling_on_sc=False),
        scratch_types=[
            pltpu.VMEM((4, _CH), jnp.int32),
            pltpu.VMEM((4, _CH), jnp.int32),
            pltpu.VMEM((2, _CH, _D), jnp.float32),
            pltpu.VMEM_SHARED((N, _D), jnp.float32),
            pltpu.VMEM_SHARED((n_pad, _D), jnp.float32),
            pltpu.SemaphoreType.DMA,
            pltpu.SemaphoreType.DMA,
            pltpu.SemaphoreType.DMA,
        ],
    )
    return f(h, ei, zeros_init)


_BR = 6250


def _sigmoid(x):
    return 1.0 / (1.0 + jnp.exp(-x))


def _tc_update(h128, parts128, Wb, bb, eps2d):
    R = h128.shape[0]
    nb = R // _BR

    def ubody(eps_ref, h_ref, p0_ref, p1_ref, W_ref, b_ref, o_ref):
        t = h_ref[...] * (1.0 + eps_ref[0, 0]) + p0_ref[0] + p1_ref[0]
        o_ref[...] = jnp.maximum(
            jnp.dot(t, W_ref[...], preferred_element_type=jnp.float32)
            + b_ref[...], 0.0)

    return pl.pallas_call(
        ubody,
        grid=(nb,),
        in_specs=[
            pl.BlockSpec(memory_space=pltpu.SMEM),
            pl.BlockSpec((_BR, 128), lambda i: (i, 0)),
            pl.BlockSpec((1, _BR, 128), lambda i: (0, i, 0)),
            pl.BlockSpec((1, _BR, 128), lambda i: (1, i, 0)),
            pl.BlockSpec((128, 128), lambda i: (0, 0)),
            pl.BlockSpec((1, 128), lambda i: (0, 0)),
        ],
        out_specs=pl.BlockSpec((_BR, 128), lambda i: (i, 0)),
        out_shape=jax.ShapeDtypeStruct((R, 128), jnp.float32),
    )(eps2d, h128, parts128, parts128, Wb, bb)


def _tc_final(h128, parts128, gWb, gbb, eps2d, W1b, b1b, W2b, b2b, W3b, b3b):
    R = h128.shape[0]
    nb = R // _BR

    def fbody(eps_ref, h_ref, p0_ref, p1_ref, gW_ref, gb_ref,
              W1_ref, b1_ref, W2_ref, b2_ref, W3_ref, b3_ref, o_ref):
        t = h_ref[...] * (1.0 + eps_ref[0, 0]) + p0_ref[0] + p1_ref[0]
        t = jnp.maximum(
            jnp.dot(t, gW_ref[...], preferred_element_type=jnp.float32)
            + gb_ref[...], 0.0)
        o = _sigmoid(jnp.dot(t, W1_ref[...],
                             preferred_element_type=jnp.float32) + b1_ref[...])
        o = _sigmoid(jnp.dot(o, W2_ref[...],
                             preferred_element_type=jnp.float32) + b2_ref[...])
        o_ref[...] = _sigmoid(jnp.dot(o, W3_ref[...],
                                      preferred_element_type=jnp.float32)
                              + b3_ref[...])

    return pl.pallas_call(
        fbody,
        grid=(nb,),
        in_specs=[
            pl.BlockSpec(memory_space=pltpu.SMEM),
            pl.BlockSpec((_BR, 128), lambda i: (i, 0)),
            pl.BlockSpec((1, _BR, 128), lambda i: (0, i, 0)),
            pl.BlockSpec((1, _BR, 128), lambda i: (1, i, 0)),
            pl.BlockSpec((128, 128), lambda i: (0, 0)),
            pl.BlockSpec((1, 128), lambda i: (0, 0)),
            pl.BlockSpec((128, 320), lambda i: (0, 0)),
            pl.BlockSpec((1, 320), lambda i: (0, 0)),
            pl.BlockSpec((320, 480), lambda i: (0, 0)),
            pl.BlockSpec((1, 480), lambda i: (0, 0)),
            pl.BlockSpec((480, 16), lambda i: (0, 0)),
            pl.BlockSpec((1, 16), lambda i: (0, 0)),
        ],
        out_specs=pl.BlockSpec((_BR, 16), lambda i: (i, 0)),
        out_shape=jax.ShapeDtypeStruct((R, 16), jnp.float32),
    )(eps2d, h128, parts128, parts128, gWb, gbb,
      W1b, b1b, W2b, b2b, W3b, b3b)


def kernel(x, edge_index, gin_W1, gin_b1, gin_W2, gin_b2, gin_W3, gin_b3,
           gin_eps, W1, b1, W2, b2, W3, b3):
    N = x.shape[0]
    E = edge_index.shape[1]
    ei = edge_index.astype(jnp.int32)
    assert E % (_NW * 4 * _CH) == 0, "edge count must split evenly"
    assert N % (16 * _BR) == 0
    n_pad = -(-(N + 1) // 128) * 128
    zeros_init = jnp.zeros((n_pad, _D), jnp.float32)
    R = N // 16
    eye16 = jnp.eye(16, dtype=jnp.float32)

    dpad = _D - x.shape[1]
    gWbs = tuple(jnp.kron(eye16, jnp.pad(W, ((0, dpad), (0, dpad))))
                 for W in (gin_W1, gin_W2, gin_W3))
    gbbs = tuple(jnp.tile(jnp.pad(b, (0, dpad)), 16).reshape(1, 128)
                 for b in (gin_b1, gin_b2, gin_b3))
    W1b = jnp.kron(eye16, jnp.pad(W1, ((0, dpad), (0, 0))))
    b1b = jnp.tile(b1, 16).reshape(1, 320)
    W2b = jnp.kron(eye16, W2)
    b2b = jnp.tile(b2, 16).reshape(1, 480)
    W3b = jnp.kron(eye16, W3)
    b3b = jnp.tile(b3, 16).reshape(1, 16)

    h = jnp.pad(x, ((0, 0), (0, dpad)))
    for l in range(2):
        parts = _sc_agg(h, ei, zeros_init)
        h128 = _tc_update(h.reshape(R, 128), parts.reshape(2, R, 128),
                          gWbs[l], gbbs[l], gin_eps[l].reshape(1, 1))
        h = h128.reshape(N, _D)
    parts = _sc_agg(h, ei, zeros_init)
    out16 = _tc_final(h.reshape(R, 128), parts.reshape(2, R, 128),
                      gWbs[2], gbbs[2], gin_eps[2].reshape(1, 1),
                      W1b, b1b, W2b, b2b, W3b, b3b)
    return out16.reshape(N, 1)
